# baseline (speedup 1.0000x reference)
import functools
import math

import jax
import jax.numpy as jnp
import numpy as np
from jax import lax
from jax.experimental import pallas as pl
from jax.experimental.pallas import tpu as pltpu
from jax.experimental.pallas import tpu_sc as plsc

H = 128
T = 4
EDGE_IN = 16
DEG = 16
NUM_CLASS = 16

_NB = 200
_NB_POST = 200
_CH = 128


def _sc_mesh():
    info = plsc.get_sparse_core_info()
    nw = info.num_cores * info.num_subcores
    mesh = plsc.VectorSubcoreMesh(core_axis_name="c", subcore_axis_name="s")
    return info, nw, mesh


def _wid(info):
    return lax.axis_index("s") * info.num_cores + lax.axis_index("c")


def _my_chunks(n_chunks, wid, nw):
    return (n_chunks - wid + nw - 1) // nw


def _sc_count(el):
    E = el.shape[0]
    n_chunks = E // _CH
    info, nw, mesh = _sc_mesh()
    i32 = jnp.int32

    @functools.partial(
        pl.kernel,
        out_type=jax.ShapeDtypeStruct((nw * 64,), i32),
        mesh=mesh,
        scratch_types=[
            pltpu.VMEM((_CH,), i32),
            pltpu.VMEM((64,), i32),
        ],
    )
    def k(el_hbm, cnt_hbm, elc_v, acc_v):
        wid = _wid(info)
        zero16 = jnp.zeros((16,), i32)

        def body(t, carry):
            c = wid + t * nw
            pltpu.sync_copy(el_hbm.at[pl.ds(c * _CH, _CH)], elc_v)
            a0, a1, a2, a3 = carry
            for j in range(_CH // 16):
                v = jnp.clip(elc_v[pl.ds(j * 16, 16)], 1, 4)
                a0 = a0 + (v == 1).astype(i32)
                a1 = a1 + (v == 2).astype(i32)
                a2 = a2 + (v == 3).astype(i32)
                a3 = a3 + (v == 4).astype(i32)
            return (a0, a1, a2, a3)

        acc = lax.fori_loop(0, _my_chunks(n_chunks, wid, nw), body,
                            (zero16, zero16, zero16, zero16))
        for kb in range(4):
            acc_v[pl.ds(kb * 16, 16)] = acc[kb]
        pltpu.sync_copy(acc_v, cnt_hbm.at[pl.ds(wid * 64, 64)])

    return k(el)


def _sc_pos_scatter(el, counts, efdt):
    E = el.shape[0]
    n_chunks = E // _CH
    info, nw, mesh = _sc_mesh()
    i32 = jnp.int32

    @functools.partial(
        pl.kernel,
        out_type=(
            jax.ShapeDtypeStruct((E,), i32),
            jax.ShapeDtypeStruct((16,), i32),
        ),
        mesh=mesh,
        scratch_types=[
            pltpu.VMEM((_CH,), i32),
            pltpu.VMEM((_CH,), i32),
            pltpu.VMEM((1, _CH), i32),
            pltpu.VMEM((nw * 64,), i32),
            pltpu.VMEM((16,), i32),
            pltpu.VMEM((_CH, 128), jnp.float32),
            pltpu.SemaphoreType.DMA,
        ],
    )
    def k(el_hbm, cnt_hbm, efdt_hbm, inv_hbm, bases_hbm,
          elc_v, inv_v, inv2d, allc_v, base_loc, rows_v, sem):
        wid = _wid(info)
        zero16 = jnp.zeros((16,), i32)
        pltpu.sync_copy(cnt_hbm, allc_v)

        tot = [zero16, zero16, zero16, zero16]
        pre = [zero16, zero16, zero16, zero16]
        for w in range(nw):
            mlt = (w < wid).astype(i32)
            for kb in range(4):
                v = allc_v[pl.ds(w * 64 + kb * 16, 16)]
                tot[kb] = tot[kb] + v
                pre[kb] = pre[kb] + v * mlt
        t1 = jnp.sum(tot[0])
        t2 = jnp.sum(tot[1])
        t3 = jnp.sum(tot[2])
        b2 = t1
        b3 = t1 + t2
        b4 = t1 + t2 + t3
        bases = [jnp.int32(0), b2, b3, b4]
        offs0 = tuple(bases[kb] + jnp.sum(pre[kb]) for kb in range(4))

        @pl.when(wid == 0)
        def _():
            lane = lax.iota(i32, 16)
            bv = ((lane == 1).astype(i32) * b2
                  + (lane == 2).astype(i32) * b3
                  + (lane == 3).astype(i32) * b4)
            base_loc[...] = bv
            pltpu.sync_copy(base_loc, bases_hbm)

        def body(t, offs):
            c = wid + t * nw
            base = c * _CH
            pltpu.sync_copy(el_hbm.at[pl.ds(base, _CH)], elc_v)
            o0, o1, o2, o3 = offs
            offs_l = [o0, o1, o2, o3]
            for j in range(_CH // 16):
                v = jnp.clip(elc_v[pl.ds(j * 16, 16)], 1, 4)
                pos = zero16
                for kb in range(4):
                    mk = v == (kb + 1)
                    ck = plsc.cumsum(mk.astype(i32))
                    pos = jnp.where(mk, offs_l[kb] + ck - 1, pos)
                    offs_l[kb] = offs_l[kb] + jnp.sum(mk.astype(i32))
                inv_v[pl.ds(j * 16, 16)] = pos
            pltpu.sync_copy(inv_v, inv_hbm.at[pl.ds(base, _CH)])
            return (offs_l[0], offs_l[1], offs_l[2], offs_l[3])

        lax.fori_loop(0, _my_chunks(n_chunks, wid, nw), body, offs0)

    return k(el, counts, efdt)


def _sc_gather(table, idx):
    E = idx.shape[0]
    D = table.shape[1]
    n_chunks = E // _CH
    assert n_chunks * _CH == E
    info, nw, mesh = _sc_mesh()

    @functools.partial(
        pl.kernel,
        out_type=jax.ShapeDtypeStruct((E, D), table.dtype),
        mesh=mesh,
        scratch_types=[
            pltpu.VMEM((_CH,), jnp.int32),
            pltpu.VMEM((_CH, D), jnp.float32),
            pltpu.SemaphoreType.DMA,
        ],
    )
    def k(table_hbm, idx_hbm, out_hbm, idx_v, rows_v, sem):
        wid = _wid(info)

        def body(t, carry):
            base = (wid + t * nw) * _CH
            pltpu.sync_copy(idx_hbm.at[pl.ds(base, _CH)], idx_v)
            pltpu.async_copy(table_hbm.at[idx_v], rows_v, sem).wait()
            pltpu.sync_copy(rows_v, out_hbm.at[pl.ds(base, _CH)])
            return carry

        lax.fori_loop(0, _my_chunks(n_chunks, wid, nw), body, 0)

    return k(table, idx)


def _sc_gather2(ta, tb, idx):
    E = idx.shape[0]
    n_chunks = E // _CH
    info, nw, mesh = _sc_mesh()

    @functools.partial(
        pl.kernel,
        out_type=(
            jax.ShapeDtypeStruct(ta.shape, jnp.float32),
            jax.ShapeDtypeStruct(tb.shape, jnp.float32),
        ),
        mesh=mesh,
        scratch_types=[
            pltpu.VMEM((_CH,), jnp.int32),
            pltpu.VMEM((_CH, ta.shape[1]), jnp.float32),
            pltpu.VMEM((_CH, tb.shape[1]), jnp.float32),
            pltpu.SemaphoreType.DMA,
            pltpu.SemaphoreType.DMA,
        ],
    )
    def k(ta_hbm, tb_hbm, idx_hbm, oa_hbm, ob_hbm,
          idx_v, ra_v, rb_v, sem1, sem2):
        wid = _wid(info)

        def body(t, carry):
            base = (wid + t * nw) * _CH
            pltpu.sync_copy(idx_hbm.at[pl.ds(base, _CH)], idx_v)
            pltpu.async_copy(ta_hbm.at[idx_v], ra_v, sem1).wait()
            pltpu.sync_copy(ra_v, oa_hbm.at[pl.ds(base, _CH)])
            pltpu.async_copy(tb_hbm.at[idx_v], rb_v, sem2).wait()
            pltpu.sync_copy(rb_v, ob_hbm.at[pl.ds(base, _CH)])
            return carry

        lax.fori_loop(0, _my_chunks(n_chunks, wid, nw), body, 0)

    return k(ta, tb, idx)


def _dot(a, b):
    bf = jnp.bfloat16
    return jnp.dot(a.astype(bf), b.astype(bf), preferred_element_type=jnp.float32)


def _q_body(nf_ref, w_ref, b_ref, q_ref):
    q_ref[...] = _dot(nf_ref[...], w_ref[...]) + b_ref[...]


def _lstm_body(efdt_ref, bases_smem,
               wx_ref, wh1_ref, wh2_ref, b1_ref, b2_ref,
               wd1_ref, wd2_ref, bd1_ref, bd2_ref, eow2_ref,
               me_ref, h2o_ref,
               h1_r, c1_r, h2_r, c2_r, h1s_r, h2s_r, *, blk):
    f32 = jnp.float32
    i32 = jnp.int32
    pid = pl.program_id(0)
    pos_last = (pid + 1) * blk - 1
    nsteps = (1
              + (pos_last >= bases_smem[1]).astype(i32)
              + (pos_last >= bases_smem[2]).astype(i32)
              + (pos_last >= bases_smem[3]).astype(i32))

    efdt = efdt_ref[...]
    ef = efdt[:, 0:T * EDGE_IN]
    lenf = jnp.clip(efdt[:, T * EDGE_IN + T:T * EDGE_IN + T + 1], 1.0, 4.0)
    g_all = 1.0 / jnp.log(f32(np.e) + efdt[:, T * EDGE_IN:T * EDGE_IN + T])

    wx = wx_ref[...]
    b1 = b1_ref[...]
    b2 = b2_ref[...]
    bd1 = bd1_ref[...]
    bd2 = bd2_ref[...]

    zx = _dot(ef[:, 0:EDGE_IN], wx)
    g = g_all[:, 0:1]
    sel = (lenf == 1.0).astype(f32)

    cadj1 = jnp.tanh(bd1) * (g - 1.0)
    z1 = zx[:, :4 * H] + b1
    c1 = (jax.nn.sigmoid(z1[:, H:2 * H]) * cadj1
          + jax.nn.sigmoid(z1[:, 0:H]) * jnp.tanh(z1[:, 3 * H:]))
    h1 = jax.nn.sigmoid(z1[:, 2 * H:3 * H]) * jnp.tanh(c1)
    h1_r[...] = h1
    c1_r[...] = c1
    h1s_r[...] = h1 * sel

    cadj2 = jnp.tanh(bd2) * (g - 1.0)
    z2 = zx[:, 4 * H:] + b2
    c2 = (jax.nn.sigmoid(z2[:, H:2 * H]) * cadj2
          + jax.nn.sigmoid(z2[:, 0:H]) * jnp.tanh(z2[:, 3 * H:]))
    h2 = jax.nn.sigmoid(z2[:, 2 * H:3 * H]) * jnp.tanh(c2)
    h2_r[...] = h2
    c2_r[...] = c2
    h2s_r[...] = h2 * sel

    for t in range(1, T):

        @pl.when(nsteps > t)
        def _(t=t):
            x_t = ef[:, t * EDGE_IN:(t + 1) * EDGE_IN]
            zx_t = _dot(x_t, wx)
            g_t = g_all[:, t:t + 1]
            sel_t = (lenf == f32(t + 1)).astype(f32)

            c1p = c1_r[...]
            h1p = h1_r[...]
            cs1 = jnp.tanh(_dot(c1p, wd1_ref[...]) + bd1)
            cadj1t = c1p - cs1 + cs1 * g_t
            z1t = zx_t[:, :4 * H] + _dot(h1p, wh1_ref[...]) + b1
            c1n = (jax.nn.sigmoid(z1t[:, H:2 * H]) * cadj1t
                   + jax.nn.sigmoid(z1t[:, 0:H]) * jnp.tanh(z1t[:, 3 * H:]))
            h1n = jax.nn.sigmoid(z1t[:, 2 * H:3 * H]) * jnp.tanh(c1n)
            c1_r[...] = c1n
            h1_r[...] = h1n
            h1s_r[...] = h1s_r[...] + h1n * sel_t

            c2p = c2_r[...]
            h2p = h2_r[...]
            cs2 = jnp.tanh(_dot(c2p, wd2_ref[...]) + bd2)
            cadj2t = c2p - cs2 + cs2 * g_t
            z2t = zx_t[:, 4 * H:] + _dot(h2p, wh2_ref[...]) + b2
            c2n = (jax.nn.sigmoid(z2t[:, H:2 * H]) * cadj2t
                   + jax.nn.sigmoid(z2t[:, 0:H]) * jnp.tanh(z2t[:, 3 * H:]))
            h2n = jax.nn.sigmoid(z2t[:, 2 * H:3 * H]) * jnp.tanh(c2n)
            c2_r[...] = c2n
            h2_r[...] = h2n
            h2s_r[...] = h2s_r[...] + h2n * sel_t

    me_ref[...] = _dot(h1s_r[...], eow2_ref[...])
    h2o_ref[...] = h2s_r[...]


def _post_body(qs_ref, me_ref, h2o_ref, q_ref, nf_ref, attn_ref,
               nw_ref, nb_ref, fcw_ref, fcb_ref, out_ref, *, nb_nodes):
    f32 = jnp.float32
    m = jnp.maximum(qs_ref[...] + me_ref[...], 0.0)
    a = _dot(h2o_ref[...], attn_ref[...])
    a = jnp.where(a > 0, a, 0.01 * a)

    a2 = a.reshape(nb_nodes, DEG)
    z = a2 - jnp.max(a2, axis=-1, keepdims=True)
    zi = z[:, :, None]
    zj = z[:, None, :]
    jj = lax.broadcasted_iota(jnp.int32, (nb_nodes, DEG, DEG), 2)
    ii = lax.broadcasted_iota(jnp.int32, (nb_nodes, DEG, DEG), 1)
    beq = ((zj > zi) | ((zj == zi) & (jj <= ii))).astype(f32)
    p_pos = jnp.sum(beq, axis=2)
    csum = jnp.sum(beq * zj, axis=2)
    isgt = (1.0 + p_pos * z > csum).astype(f32)
    k_sup = jnp.max(isgt * p_pos, axis=-1, keepdims=True)
    s_sup = jnp.sum(isgt * z, axis=-1, keepdims=True)
    tau = (s_sup - 1.0) / k_sup
    alpha = jnp.maximum(z - tau, 0.0)

    m3 = m.reshape(nb_nodes, DEG, H)
    hagg = jnp.sum(m3 * alpha[:, :, None], axis=1)

    hr = hagg - q_ref[...]
    nw = nw_ref[...]
    act = jnp.maximum(
        _dot(nf_ref[...], nw[:H]) + _dot(hr, nw[H:]) + nb_ref[...], 0.0)
    out_ref[...] = _dot(act, fcw_ref[...]) + fcb_ref[...]


def _const_spec(shape):
    return pl.BlockSpec(shape, lambda i: (0,) * len(shape))


def kernel(node_features, src_idx, edge_features, delta_t, edge_len, params):
    p = params
    n_nodes = node_features.shape[0]
    e_edges = src_idx.shape[0]
    f32 = jnp.float32

    eob = p["eo_b"].reshape(1, H)
    eow = p["eo_W"]

    nb_q = 2000
    q = pl.pallas_call(
        _q_body,
        grid=(n_nodes // nb_q,),
        in_specs=[
            pl.BlockSpec((nb_q, H), lambda i: (i, 0)),
            _const_spec((H, H)),
            _const_spec((1, H)),
        ],
        out_specs=pl.BlockSpec((nb_q, H), lambda i: (i, 0)),
        out_shape=jax.ShapeDtypeStruct((n_nodes, H), f32),
    )(node_features, eow[:H], eob)

    ef2 = edge_features.reshape(e_edges, T * EDGE_IN)
    efdt = jnp.concatenate(
        [ef2, delta_t, edge_len.astype(f32)[:, None],
         jnp.zeros((e_edges, 128 - T * EDGE_IN - T - 1), f32)], axis=1)
    counts = _sc_count(edge_len)
    elc_dbg = jnp.clip(edge_len, 1, 4)
    order = jnp.argsort(elc_dbg, stable=True)
    invpos = jnp.zeros_like(order).at[order].set(
        jnp.arange(e_edges, dtype=jnp.int32)) + 0 * counts[0]
    bases = jnp.zeros((16,), jnp.int32)
    bases = bases.at[1].set(jnp.sum(elc_dbg == 1)).at[2].set(
        jnp.sum(elc_dbg <= 2)).at[3].set(jnp.sum(elc_dbg <= 3))
    efdtp = jnp.zeros_like(efdt).at[invpos].set(efdt)

    qsrc = _sc_gather(q, src_idx)

    nb = _NB
    blk = nb * DEG
    wx = jnp.concatenate([p["lstm1_Wx"], p["lstm2_Wx"]], axis=1)
    b1 = p["lstm1_b"].reshape(1, 4 * H)
    b2 = p["lstm2_b"].reshape(1, 4 * H)
    bd1 = p["lstm1_bd"].reshape(1, H)
    bd2 = p["lstm2_bd"].reshape(1, H)

    me_p, h2s_p = pl.pallas_call(
        functools.partial(_lstm_body, blk=blk),
        grid=(e_edges // blk,),
        in_specs=[
            pl.BlockSpec((blk, 128), lambda i: (i, 0)),
            pl.BlockSpec(memory_space=pltpu.SMEM),
            _const_spec((EDGE_IN, 8 * H)),
            _const_spec((H, 4 * H)),
            _const_spec((H, 4 * H)),
            _const_spec((1, 4 * H)),
            _const_spec((1, 4 * H)),
            _const_spec((H, H)),
            _const_spec((H, H)),
            _const_spec((1, H)),
            _const_spec((1, H)),
            _const_spec((H, H)),
        ],
        out_specs=[
            pl.BlockSpec((blk, H), lambda i: (i, 0)),
            pl.BlockSpec((blk, H), lambda i: (i, 0)),
        ],
        out_shape=[
            jax.ShapeDtypeStruct((e_edges, H), f32),
            jax.ShapeDtypeStruct((e_edges, H), f32),
        ],
        scratch_shapes=[pltpu.VMEM((blk, H), f32)] * 6,
    )(efdtp, bases,
      wx, p["lstm1_Wh"], p["lstm2_Wh"], b1, b2,
      p["lstm1_Wd"], p["lstm2_Wd"], bd1, bd2, eow[H:])

    me, h2s = _sc_gather2(me_p, h2s_p, invpos)

    nbp = _NB_POST
    blkp = nbp * DEG
    nodeb = p["node_b"].reshape(1, H)
    fcb = p["fc_b"].reshape(1, NUM_CLASS)
    out = pl.pallas_call(
        functools.partial(_post_body, nb_nodes=nbp),
        grid=(n_nodes // nbp,),
        in_specs=[
            pl.BlockSpec((blkp, H), lambda i: (i, 0)),
            pl.BlockSpec((blkp, H), lambda i: (i, 0)),
            pl.BlockSpec((blkp, H), lambda i: (i, 0)),
            pl.BlockSpec((nbp, H), lambda i: (i, 0)),
            pl.BlockSpec((nbp, H), lambda i: (i, 0)),
            _const_spec((H, 1)),
            _const_spec((2 * H, H)),
            _const_spec((1, H)),
            _const_spec((H, NUM_CLASS)),
            _const_spec((1, NUM_CLASS)),
        ],
        out_specs=pl.BlockSpec((nbp, NUM_CLASS), lambda i: (i, 0)),
        out_shape=jax.ShapeDtypeStruct((n_nodes, NUM_CLASS), f32),
    )(qsrc, me, h2s, q, node_features, p["attn_W"],
      p["node_W"], nodeb, p["fc_W"], fcb)
    return out


# speedup vs baseline: 1.9031x; 1.9031x over previous
import functools
import math

import jax
import jax.numpy as jnp
import numpy as np
from jax import lax
from jax.experimental import pallas as pl
from jax.experimental.pallas import tpu as pltpu
from jax.experimental.pallas import tpu_sc as plsc

H = 128
T = 4
EDGE_IN = 16
DEG = 16
NUM_CLASS = 16

_NB = 200
_NB_POST = 200


def _sc_gather(table, idx):
    E = idx.shape[0]
    D = table.shape[1]
    CH = 128
    n_chunks = E // CH
    assert n_chunks * CH == E
    info = plsc.get_sparse_core_info()
    nc = info.num_cores
    nw = nc * info.num_subcores
    per_w = math.ceil(n_chunks / nw)
    mesh = plsc.VectorSubcoreMesh(core_axis_name="c", subcore_axis_name="s")

    @functools.partial(
        pl.kernel,
        out_type=jax.ShapeDtypeStruct((E, D), table.dtype),
        mesh=mesh,
        scratch_types=[
            pltpu.VMEM((CH,), jnp.int32),
            pltpu.VMEM((CH, D), jnp.float32),
            pltpu.SemaphoreType.DMA,
        ],
    )
    def k(table_hbm, idx_hbm, out_hbm, idx_v, rows_v, sem):
        wid = lax.axis_index("s") * nc + lax.axis_index("c")

        def body(t, carry):
            c = wid + t * nw

            @pl.when(c < n_chunks)
            def _():
                base = c * CH
                pltpu.sync_copy(idx_hbm.at[pl.ds(base, CH)], idx_v)
                pltpu.async_copy(table_hbm.at[idx_v], rows_v, sem).wait()
                pltpu.sync_copy(rows_v, out_hbm.at[pl.ds(base, CH)])

            return carry

        lax.fori_loop(0, per_w, body, 0)

    return k(table, idx)


def _dot(a, b):
    bf = jnp.bfloat16
    return jnp.dot(a.astype(bf), b.astype(bf), preferred_element_type=jnp.float32)


def _q_body(nf_ref, w_ref, b_ref, q_ref):
    q_ref[...] = _dot(nf_ref[...], w_ref[...]) + b_ref[...]


def _lstm_body(ef_ref, dt_ref, el_ref,
               wx_ref, wh1_ref, wh2_ref, b1_ref, b2_ref,
               wd1_ref, wd2_ref, bd1_ref, bd2_ref,
               attn_ref, eow2_ref,
               me_ref, a_ref, *, blk):
    f32 = jnp.float32
    ef = ef_ref[...]
    dt = dt_ref[...]
    el = el_ref[...]
    idx_t = jnp.clip(el - 1, 0, T - 1)

    wx = wx_ref[...]
    wh1 = wh1_ref[...]
    wh2 = wh2_ref[...]
    b1 = b1_ref[...]
    b2 = b2_ref[...]
    wd1 = wd1_ref[...]
    wd2 = wd2_ref[...]
    bd1 = bd1_ref[...]
    bd2 = bd2_ref[...]

    zeros = jnp.zeros((blk, H), f32)
    h1 = zeros
    c1 = zeros
    h2 = zeros
    c2 = zeros
    h1_sel = zeros
    h2_sel = zeros
    g_all = 1.0 / jnp.log(f32(np.e) + dt)

    for t in range(T):
        x_t = ef[:, t * EDGE_IN:(t + 1) * EDGE_IN]
        zx = _dot(x_t, wx)
        g = g_all[:, t:t + 1]
        sel = (idx_t == t).astype(f32)

        cs1 = jnp.tanh(_dot(c1, wd1) + bd1)
        cadj1 = c1 - cs1 + cs1 * g
        z1 = zx[:, :4 * H] + _dot(h1, wh1) + b1
        c1 = (jax.nn.sigmoid(z1[:, H:2 * H]) * cadj1
              + jax.nn.sigmoid(z1[:, 0:H]) * jnp.tanh(z1[:, 3 * H:]))
        h1 = jax.nn.sigmoid(z1[:, 2 * H:3 * H]) * jnp.tanh(c1)
        h1_sel = h1_sel + h1 * sel

        cs2 = jnp.tanh(_dot(c2, wd2) + bd2)
        cadj2 = c2 - cs2 + cs2 * g
        z2 = zx[:, 4 * H:] + _dot(h2, wh2) + b2
        c2 = (jax.nn.sigmoid(z2[:, H:2 * H]) * cadj2
              + jax.nn.sigmoid(z2[:, 0:H]) * jnp.tanh(z2[:, 3 * H:]))
        h2 = jax.nn.sigmoid(z2[:, 2 * H:3 * H]) * jnp.tanh(c2)
        h2_sel = h2_sel + h2 * sel

    me_ref[...] = _dot(h1_sel, eow2_ref[...])
    a = _dot(h2_sel, attn_ref[...])
    a_ref[...] = jnp.where(a > 0, a, 0.01 * a)


def _post_body(qs_ref, me_ref, a_ref, q_ref, nf_ref,
               nw_ref, nb_ref, fcw_ref, fcb_ref, out_ref, *, nb_nodes):
    f32 = jnp.float32
    m = jnp.maximum(qs_ref[...] + me_ref[...], 0.0)
    a = a_ref[...]

    a2 = a.reshape(nb_nodes, DEG)
    z = a2 - jnp.max(a2, axis=-1, keepdims=True)
    zi = z[:, :, None]
    zj = z[:, None, :]
    jj = lax.broadcasted_iota(jnp.int32, (nb_nodes, DEG, DEG), 2)
    ii = lax.broadcasted_iota(jnp.int32, (nb_nodes, DEG, DEG), 1)
    beq = ((zj > zi) | ((zj == zi) & (jj <= ii))).astype(f32)
    p_pos = jnp.sum(beq, axis=2)
    csum = jnp.sum(beq * zj, axis=2)
    isgt = (1.0 + p_pos * z > csum).astype(f32)
    k_sup = jnp.max(isgt * p_pos, axis=-1, keepdims=True)
    s_sup = jnp.sum(isgt * z, axis=-1, keepdims=True)
    tau = (s_sup - 1.0) / k_sup
    alpha = jnp.maximum(z - tau, 0.0)

    m3 = m.reshape(nb_nodes, DEG, H)
    hagg = jnp.sum(m3 * alpha[:, :, None], axis=1)

    hr = hagg - q_ref[...]
    nw = nw_ref[...]
    act = jnp.maximum(
        _dot(nf_ref[...], nw[:H]) + _dot(hr, nw[H:]) + nb_ref[...], 0.0)
    out_ref[...] = _dot(act, fcw_ref[...]) + fcb_ref[...]


def _const_spec(shape):
    return pl.BlockSpec(shape, lambda i: (0,) * len(shape))


def kernel(node_features, src_idx, edge_features, delta_t, edge_len, params):
    p = params
    n_nodes = node_features.shape[0]
    e_edges = src_idx.shape[0]

    eob = p["eo_b"].reshape(1, H)
    eow = p["eo_W"]

    nb_q = 2000
    q = pl.pallas_call(
        _q_body,
        grid=(n_nodes // nb_q,),
        in_specs=[
            pl.BlockSpec((nb_q, H), lambda i: (i, 0)),
            _const_spec((H, H)),
            _const_spec((1, H)),
        ],
        out_specs=pl.BlockSpec((nb_q, H), lambda i: (i, 0)),
        out_shape=jax.ShapeDtypeStruct((n_nodes, H), jnp.float32),
    )(node_features, eow[:H], eob)

    qsrc = _sc_gather(q, src_idx)

    nb = _NB
    blk = nb * DEG
    ef2 = edge_features.reshape(e_edges, T * EDGE_IN)
    el2 = edge_len.reshape(e_edges, 1)
    wx = jnp.concatenate([p["lstm1_Wx"], p["lstm2_Wx"]], axis=1)
    b1 = p["lstm1_b"].reshape(1, 4 * H)
    b2 = p["lstm2_b"].reshape(1, 4 * H)
    bd1 = p["lstm1_bd"].reshape(1, H)
    bd2 = p["lstm2_bd"].reshape(1, H)

    me, a = pl.pallas_call(
        functools.partial(_lstm_body, blk=blk),
        grid=(e_edges // blk,),
        in_specs=[
            pl.BlockSpec((blk, T * EDGE_IN), lambda i: (i, 0)),
            pl.BlockSpec((blk, T), lambda i: (i, 0)),
            pl.BlockSpec((blk, 1), lambda i: (i, 0)),
            _const_spec((EDGE_IN, 8 * H)),
            _const_spec((H, 4 * H)),
            _const_spec((H, 4 * H)),
            _const_spec((1, 4 * H)),
            _const_spec((1, 4 * H)),
            _const_spec((H, H)),
            _const_spec((H, H)),
            _const_spec((1, H)),
            _const_spec((1, H)),
            _const_spec((H, 1)),
            _const_spec((H, H)),
        ],
        out_specs=[
            pl.BlockSpec((blk, H), lambda i: (i, 0)),
            pl.BlockSpec((blk, 1), lambda i: (i, 0)),
        ],
        out_shape=[
            jax.ShapeDtypeStruct((e_edges, H), jnp.float32),
            jax.ShapeDtypeStruct((e_edges, 1), jnp.float32),
        ],
    )(ef2, delta_t, el2,
      wx, p["lstm1_Wh"], p["lstm2_Wh"], b1, b2,
      p["lstm1_Wd"], p["lstm2_Wd"], bd1, bd2,
      p["attn_W"], eow[H:])

    nbp = _NB_POST
    blkp = nbp * DEG
    nodeb = p["node_b"].reshape(1, H)
    fcb = p["fc_b"].reshape(1, NUM_CLASS)
    out = pl.pallas_call(
        functools.partial(_post_body, nb_nodes=nbp),
        grid=(n_nodes // nbp,),
        in_specs=[
            pl.BlockSpec((blkp, H), lambda i: (i, 0)),
            pl.BlockSpec((blkp, H), lambda i: (i, 0)),
            pl.BlockSpec((blkp, 1), lambda i: (i, 0)),
            pl.BlockSpec((nbp, H), lambda i: (i, 0)),
            pl.BlockSpec((nbp, H), lambda i: (i, 0)),
            _const_spec((2 * H, H)),
            _const_spec((1, H)),
            _const_spec((H, NUM_CLASS)),
            _const_spec((1, NUM_CLASS)),
        ],
        out_specs=pl.BlockSpec((nbp, NUM_CLASS), lambda i: (i, 0)),
        out_shape=jax.ShapeDtypeStruct((n_nodes, NUM_CLASS), jnp.float32),
    )(qsrc, me, a, q, node_features,
      p["node_W"], nodeb, p["fc_W"], fcb)
    return out
